# baseline (speedup 1.0000x reference)
import functools

import jax
import jax.numpy as jnp
from jax import lax
from jax.experimental import pallas as pl
from jax.experimental.pallas import tpu as pltpu
from jax.experimental.pallas import tpu_sc as plsc

F32 = jnp.float32

NC = 2
NS = 16
NW = NC * NS


def _edge_enc_body(elo_ref, ehi_ref, w_ref, b_ref, out_ref):
    def enc(block):
        return jax.nn.relu(
            lax.dot_general(block, w_ref[...], (((0,), (0,)), ((), ())),
                            preferred_element_type=F32)
            + b_ref[...]
        )

    lo = enc(elo_ref[...]).astype(jnp.bfloat16)
    hi = enc(ehi_ref[...]).astype(jnp.bfloat16)
    lo16 = lax.bitcast_convert_type(lo, jnp.uint16).astype(jnp.uint32)
    hi16 = lax.bitcast_convert_type(hi, jnp.uint16).astype(jnp.uint32)
    out_ref[...] = lo16 | (hi16 << 16)


def _edge_encoder(edge_attr, We0, be0, E, DE, D):
    ea_t = edge_attr.T
    TE = 3200
    grid = (E // 2) // TE
    return pl.pallas_call(
        _edge_enc_body,
        grid=(grid,),
        in_specs=[
            pl.BlockSpec((DE, TE), lambda i: (0, i)),
            pl.BlockSpec((DE, TE), lambda i, g=grid: (0, i + g)),
            pl.BlockSpec((DE, D), lambda i: (0, 0)),
            pl.BlockSpec((1, D), lambda i: (0, 0)),
        ],
        out_specs=pl.BlockSpec((TE, D), lambda i: (i, 0)),
        out_shape=jax.ShapeDtypeStruct((E // 2, D), jnp.uint32),
    )(ea_t, ea_t, We0, be0.reshape(1, D))


def _make_sc_scatter(N, E, D, with_e):
    EPT = E // NW
    CH = 80
    HCH = CH // 2
    NCHUNK = EPT // CH
    DUMP_T = 10
    RPO = N // DUMP_T

    def body(*refs):
        if with_e:
            (e_hbm, x_hbm, ei_hbm, out_hbm,
             srcall, idxd, erows, grows, acc, sems) = refs
        else:
            (x_hbm, ei_hbm, out_hbm,
             srcall, idxd, grows, acc, sems) = refs
            e_hbm = None
            erows = None

        c = lax.axis_index("c")
        s = lax.axis_index("s")

        def zrow(i, carry):
            for k in range(D // 16):
                grows[0, i, pl.ds(k * 16, 16)] = jnp.zeros((16,), F32)
            return carry

        lax.fori_loop(0, CH, zrow, 0)

        @pl.when(s < DUMP_T)
        def _():
            for j in range(RPO // CH):
                pltpu.sync_copy(grows.at[0],
                                acc.at[pl.ds(s * RPO + j * CH, CH)])
            rem = RPO % CH
            if rem:
                pltpu.sync_copy(
                    grows.at[0].at[pl.ds(0, rem)],
                    acc.at[pl.ds(s * RPO + (RPO // CH) * CH, rem)])

        wid = c * NS + s
        if with_e:
            lo = wid * (EPT // 2)
            pltpu.sync_copy(ei_hbm.at[pl.ds(lo, EPT // 2)],
                            srcall.at[pl.ds(0, EPT // 2)])
            pltpu.sync_copy(ei_hbm.at[pl.ds(E // 2 + lo, EPT // 2)],
                            srcall.at[pl.ds(EPT // 2, EPT // 2)])
        else:
            ebase = wid * EPT
            pltpu.sync_copy(ei_hbm.at[pl.ds(ebase, EPT)], srcall)
        plsc.subcore_barrier()

        if with_e:
            def start(ci, b):
                pb = lo + ci * HCH
                pltpu.async_copy(ei_hbm.at[pl.ds(E + pb, HCH)],
                                 idxd.at[b, pl.ds(0, HCH)], sems.at[b])
                pltpu.async_copy(ei_hbm.at[pl.ds(E + E // 2 + pb, HCH)],
                                 idxd.at[b, pl.ds(HCH, HCH)], sems.at[b])
                pltpu.async_copy(e_hbm.at[pl.ds(pb, HCH)], erows.at[b],
                                 sems.at[b])
                pltpu.async_copy(
                    x_hbm.at[srcall.at[pl.ds(ci * HCH, HCH)]],
                    grows.at[b, pl.ds(0, HCH)], sems.at[b])
                pltpu.async_copy(
                    x_hbm.at[srcall.at[pl.ds(EPT // 2 + ci * HCH, HCH)]],
                    grows.at[b, pl.ds(HCH, HCH)], sems.at[b])

            def finish(b):
                pltpu.make_async_copy(ei_hbm.at[pl.ds(0, CH)], idxd.at[b],
                                      sems.at[b]).wait()
                pltpu.make_async_copy(e_hbm.at[pl.ds(0, HCH)], erows.at[b],
                                      sems.at[b]).wait()
                pltpu.make_async_copy(x_hbm.at[pl.ds(0, CH)], grows.at[b],
                                      sems.at[b]).wait()

                def fold(i2, carry):
                    for g in range(D // 16):
                        w16 = erows[b, i2, pl.ds(16 * g, 16)]
                        ea = lax.bitcast_convert_type(w16 << 16, F32)
                        eb = lax.bitcast_convert_type(
                            w16 & jnp.uint32(0xFFFF0000), F32)
                        plsc.addupdate(
                            grows.at[b, i2, pl.ds(16 * g, 16)], ea)
                        plsc.addupdate(
                            grows.at[b, HCH + i2, pl.ds(16 * g, 16)], eb)
                    return carry

                lax.fori_loop(0, HCH, fold, 0)
                pltpu.sync_copy(grows.at[b], acc.at[idxd.at[b]], add=True)
        else:
            def start(ci, b):
                hb = ebase + ci * CH
                pltpu.async_copy(ei_hbm.at[pl.ds(E + hb, CH)], idxd.at[b],
                                 sems.at[b])
                pltpu.async_copy(x_hbm.at[srcall.at[pl.ds(ci * CH, CH)]],
                                 grows.at[b], sems.at[b])

            def finish(b):
                pltpu.make_async_copy(ei_hbm.at[pl.ds(0, CH)], idxd.at[b],
                                      sems.at[b]).wait()
                pltpu.make_async_copy(x_hbm.at[pl.ds(0, CH)], grows.at[b],
                                      sems.at[b]).wait()
                pltpu.sync_copy(grows.at[b], acc.at[idxd.at[b]], add=True)

        start(0, 0)

        def chunk(ci, carry):
            b = lax.rem(ci, 2)

            @pl.when(ci + 1 < NCHUNK)
            def _():
                start(ci + 1, 1 - b)

            finish(b)
            return carry

        lax.fori_loop(0, NCHUNK, chunk, 0)
        plsc.subcore_barrier()

        @pl.when(s < DUMP_T)
        def _():
            rb = s * RPO
            pltpu.sync_copy(acc.at[pl.ds(rb, RPO)],
                            out_hbm.at[c, pl.ds(rb, RPO)])

    scratch = [
        pltpu.VMEM((EPT,), jnp.int32),
        pltpu.VMEM((2, CH), jnp.int32),
    ]
    if with_e:
        scratch.append(pltpu.VMEM((2, HCH, D), jnp.uint32))
    scratch += [
        pltpu.VMEM((2, CH, D), F32),
        pltpu.VMEM_SHARED((N, D), F32),
        pltpu.SemaphoreType.DMA((2,)),
    ]

    return functools.partial(
        pl.kernel,
        out_type=jax.ShapeDtypeStruct((NC, N, D), F32),
        mesh=plsc.VectorSubcoreMesh(core_axis_name="c", subcore_axis_name="s"),
        scratch_types=scratch,
    )(body)


def _mlp1_body(parts_ref, x_ref, w1_ref, b1_ref, w2_ref, b2_ref,
               h1_ref, agg_ref, r_ref):
    agg = parts_ref[0] + parts_ref[1]
    u = jax.nn.relu(
        jnp.dot(agg, w1_ref[...], preferred_element_type=F32) + b1_ref[...])
    r = jax.nn.relu(
        jnp.dot(u, w2_ref[...], preferred_element_type=F32) + b2_ref[...])
    agg_ref[...] = agg
    r_ref[...] = r
    h1_ref[...] = x_ref[...] + r


def _mlp1(parts, x, A1, a1, A2, a2, N, D, H):
    TN = 1000
    grid = N // TN
    return pl.pallas_call(
        _mlp1_body,
        grid=(grid,),
        in_specs=[
            pl.BlockSpec((NC, TN, D), lambda i: (0, i, 0)),
            pl.BlockSpec((TN, D), lambda i: (i, 0)),
            pl.BlockSpec((D, H), lambda i: (0, 0)),
            pl.BlockSpec((1, H), lambda i: (0, 0)),
            pl.BlockSpec((H, D), lambda i: (0, 0)),
            pl.BlockSpec((1, D), lambda i: (0, 0)),
        ],
        out_specs=[
            pl.BlockSpec((TN, D), lambda i: (i, 0)),
            pl.BlockSpec((TN, D), lambda i: (i, 0)),
            pl.BlockSpec((TN, D), lambda i: (i, 0)),
        ],
        out_shape=[
            jax.ShapeDtypeStruct((N, D), F32),
            jax.ShapeDtypeStruct((N, D), F32),
            jax.ShapeDtypeStruct((N, D), F32),
        ],
    )(parts, x, A1, a1.reshape(1, H), A2, a2.reshape(1, D))


def _mlp2_body(ngrid, G, h1_ref, agg1_ref, parts_ref, gid_ref,
               w1_ref, b1_ref, w2_ref, b2_ref,
               p1_ref, pb1_ref, p2_ref, pb2_ref,
               out_ref, pooled_ref):
    i = pl.program_id(0)
    agg2 = agg1_ref[...] + parts_ref[0] + parts_ref[1]
    u = jax.nn.relu(
        jnp.dot(agg2, w1_ref[...], preferred_element_type=F32) + b1_ref[...])
    r2 = jax.nn.relu(
        jnp.dot(u, w2_ref[...], preferred_element_type=F32) + b2_ref[...])
    h2 = h1_ref[...] + r2
    gid = gid_ref[0]
    onehot = (gid == lax.broadcasted_iota(jnp.int32, (G, gid.shape[1]), 0))
    onehot = onehot.astype(F32)

    @pl.when(i == 0)
    def _():
        pooled_ref[...] = jnp.zeros_like(pooled_ref)

    pooled_ref[...] += jnp.dot(onehot, h2, preferred_element_type=F32)

    @pl.when(i == ngrid - 1)
    def _():
        pooled = pooled_ref[...]
        v = jax.nn.relu(
            jnp.dot(pooled, p1_ref[...], preferred_element_type=F32)
            + pb1_ref[...])
        out_ref[...] = (
            jnp.dot(v, p2_ref[...], preferred_element_type=F32) + pb2_ref[...])


def _mlp2_pool(h1, agg1, parts, graph_ids, B1, b1, B2, b2, P1, p1, P2, p2,
               N, D, H, G):
    TN = 1000
    grid = N // TN
    gid3 = graph_ids.reshape(grid, 1, TN)
    return pl.pallas_call(
        functools.partial(_mlp2_body, grid, G),
        grid=(grid,),
        in_specs=[
            pl.BlockSpec((TN, D), lambda i: (i, 0)),
            pl.BlockSpec((TN, D), lambda i: (i, 0)),
            pl.BlockSpec((NC, TN, D), lambda i: (0, i, 0)),
            pl.BlockSpec((1, 1, TN), lambda i: (i, 0, 0)),
            pl.BlockSpec((D, H), lambda i: (0, 0)),
            pl.BlockSpec((1, H), lambda i: (0, 0)),
            pl.BlockSpec((H, D), lambda i: (0, 0)),
            pl.BlockSpec((1, D), lambda i: (0, 0)),
            pl.BlockSpec((D, D), lambda i: (0, 0)),
            pl.BlockSpec((1, D), lambda i: (0, 0)),
            pl.BlockSpec((D, D), lambda i: (0, 0)),
            pl.BlockSpec((1, D), lambda i: (0, 0)),
        ],
        out_specs=pl.BlockSpec((G, D), lambda i: (0, 0)),
        out_shape=jax.ShapeDtypeStruct((G, D), F32),
        scratch_shapes=[pltpu.VMEM((G, D), F32)],
    )(h1, agg1, parts, gid3, B1, b1.reshape(1, H), B2, b2.reshape(1, D),
      P1, p1.reshape(1, D), P2, p2.reshape(1, D))


def kernel(x, edge_attr, We0, be0, A1, a1, A2, a2, B1, b1, B2, b2,
           P1, p1, P2, p2, edge_index, graph_ids):
    N, D = x.shape
    E, DE = edge_attr.shape
    H = A1.shape[1]
    G = 256

    ei_flat = edge_index.reshape(2 * E)
    e = _edge_encoder(edge_attr, We0, be0, E, DE, D)
    parts1 = _make_sc_scatter(N, E, D, with_e=True)(e, x, ei_flat)
    h1, agg1, r1 = _mlp1(parts1, x, A1, a1, A2, a2, N, D, H)
    parts2 = _make_sc_scatter(N, E, D, with_e=False)(r1, ei_flat)
    out = _mlp2_pool(h1, agg1, parts2, graph_ids,
                     B1, b1, B2, b2, P1, p1, P2, p2, N, D, H, G)
    return out


# speedup vs baseline: 1.1868x; 1.1868x over previous
import functools

import jax
import jax.numpy as jnp
from jax import lax
from jax.experimental import pallas as pl
from jax.experimental.pallas import tpu as pltpu
from jax.experimental.pallas import tpu_sc as plsc

F32 = jnp.float32

NC = 2
NS = 16
NW = NC * NS


def _edge_enc_body(eat_ref, w_ref, b_ref, out_ref):
    out_ref[...] = jax.nn.relu(
        lax.dot_general(eat_ref[...], w_ref[...], (((0,), (0,)), ((), ())),
                        preferred_element_type=F32)
        + b_ref[...]
    )


def _edge_encoder(edge_attr, We0, be0, E, DE, D):
    ea_t = edge_attr.T
    TE = 6400
    grid = E // TE
    return pl.pallas_call(
        _edge_enc_body,
        grid=(grid,),
        in_specs=[
            pl.BlockSpec((DE, TE), lambda i: (0, i)),
            pl.BlockSpec((DE, D), lambda i: (0, 0)),
            pl.BlockSpec((1, D), lambda i: (0, 0)),
        ],
        out_specs=pl.BlockSpec((TE, D), lambda i: (i, 0)),
        out_shape=jax.ShapeDtypeStruct((E, D), F32),
    )(ea_t, We0, be0.reshape(1, D))


def _make_sc_scatter(N, E, D, with_e):
    EPT = E // NW
    CH = 80
    NCHUNK = EPT // CH
    DUMP_T = 10
    RPO = N // DUMP_T

    def body(*refs):
        if with_e:
            (e_hbm, x_hbm, ei_hbm, out_hbm,
             srcall, idxd, erows, grows, acc, sems, semi) = refs
        else:
            (x_hbm, ei_hbm, out_hbm,
             srcall, idxd, grows, acc, sems) = refs
            e_hbm = None
            erows = None

        c = lax.axis_index("c")
        s = lax.axis_index("s")

        def zrow(i, carry):
            for k in range(D // 16):
                grows[0, i, pl.ds(k * 16, 16)] = jnp.zeros((16,), F32)
            return carry

        lax.fori_loop(0, CH, zrow, 0)

        @pl.when(s < DUMP_T)
        def _():
            for j in range(RPO // CH):
                pltpu.sync_copy(grows.at[0],
                                acc.at[pl.ds(s * RPO + j * CH, CH)])
            rem = RPO % CH
            if rem:
                pltpu.sync_copy(
                    grows.at[0].at[pl.ds(0, rem)],
                    acc.at[pl.ds(s * RPO + (RPO // CH) * CH, rem)])

        ebase = (c * NS + s) * EPT
        if not with_e:
            pltpu.sync_copy(ei_hbm.at[pl.ds(ebase, EPT)], srcall)
        plsc.subcore_barrier()

        if with_e:
            def idx_load(cj):
                r = lax.rem(cj, 3)
                hb = ebase + cj * CH
                pltpu.async_copy(ei_hbm.at[pl.ds(hb, CH)], srcall.at[r],
                                 semi.at[r])
                pltpu.async_copy(ei_hbm.at[pl.ds(E + hb, CH)], idxd.at[r],
                                 semi.at[r])

            def rows_start(cj, b):
                r = lax.rem(cj, 3)
                hb = ebase + cj * CH
                pltpu.make_async_copy(ei_hbm.at[pl.ds(0, CH)], srcall.at[r],
                                      semi.at[r]).wait()
                pltpu.make_async_copy(ei_hbm.at[pl.ds(0, CH)], idxd.at[r],
                                      semi.at[r]).wait()
                pltpu.async_copy(e_hbm.at[pl.ds(hb, CH)], erows.at[b],
                                 sems.at[b])
                pltpu.async_copy(x_hbm.at[srcall.at[r]],
                                 grows.at[b], sems.at[b])

            def finish(ci, b):
                r = lax.rem(ci, 3)
                pltpu.make_async_copy(e_hbm.at[pl.ds(0, CH)], erows.at[b],
                                      sems.at[b]).wait()
                pltpu.make_async_copy(x_hbm.at[pl.ds(0, CH)], grows.at[b],
                                      sems.at[b]).wait()
                pltpu.sync_copy(erows.at[b], acc.at[idxd.at[r]], add=True)
                pltpu.sync_copy(grows.at[b], acc.at[idxd.at[r]], add=True)

            idx_load(0)
            idx_load(1)
            rows_start(0, 0)

            def chunk(ci, carry):
                b = lax.rem(ci, 2)

                @pl.when(ci + 2 < NCHUNK)
                def _():
                    idx_load(ci + 2)

                @pl.when(ci + 1 < NCHUNK)
                def _():
                    rows_start(ci + 1, 1 - b)

                finish(ci, b)
                return carry

            lax.fori_loop(0, NCHUNK, chunk, 0)
        else:
            def start(ci, b):
                hb = ebase + ci * CH
                pltpu.async_copy(ei_hbm.at[pl.ds(E + hb, CH)], idxd.at[b],
                                 sems.at[b])
                pltpu.async_copy(x_hbm.at[srcall.at[pl.ds(ci * CH, CH)]],
                                 grows.at[b], sems.at[b])

            def finish(b):
                pltpu.make_async_copy(ei_hbm.at[pl.ds(0, CH)], idxd.at[b],
                                      sems.at[b]).wait()
                pltpu.make_async_copy(x_hbm.at[pl.ds(0, CH)], grows.at[b],
                                      sems.at[b]).wait()
                pltpu.sync_copy(grows.at[b], acc.at[idxd.at[b]], add=True)

            start(0, 0)

            def chunk(ci, carry):
                b = lax.rem(ci, 2)

                @pl.when(ci + 1 < NCHUNK)
                def _():
                    start(ci + 1, 1 - b)

                finish(b)
                return carry

            lax.fori_loop(0, NCHUNK, chunk, 0)
        plsc.subcore_barrier()

        @pl.when(s < DUMP_T)
        def _():
            rb = s * RPO
            pltpu.sync_copy(acc.at[pl.ds(rb, RPO)],
                            out_hbm.at[c, pl.ds(rb, RPO)])

    scratch = [
        pltpu.VMEM((3, CH) if with_e else (EPT,), jnp.int32),
        pltpu.VMEM((3 if with_e else 2, CH), jnp.int32),
    ]
    if with_e:
        scratch.append(pltpu.VMEM((2, CH, D), F32))
    scratch += [
        pltpu.VMEM((2, CH, D), F32),
        pltpu.VMEM_SHARED((N, D), F32),
        pltpu.SemaphoreType.DMA((2,)),
    ]
    if with_e:
        scratch.append(pltpu.SemaphoreType.DMA((3,)))

    return functools.partial(
        pl.kernel,
        out_type=jax.ShapeDtypeStruct((NC, N, D), F32),
        mesh=plsc.VectorSubcoreMesh(core_axis_name="c", subcore_axis_name="s"),
        scratch_types=scratch,
    )(body)


def _mlp1_body(parts_ref, x_ref, w1_ref, b1_ref, w2_ref, b2_ref,
               h1_ref, agg_ref, r_ref):
    agg = parts_ref[0] + parts_ref[1]
    u = jax.nn.relu(
        jnp.dot(agg, w1_ref[...], preferred_element_type=F32) + b1_ref[...])
    r = jax.nn.relu(
        jnp.dot(u, w2_ref[...], preferred_element_type=F32) + b2_ref[...])
    agg_ref[...] = agg
    r_ref[...] = r
    h1_ref[...] = x_ref[...] + r


def _mlp1(parts, x, A1, a1, A2, a2, N, D, H):
    TN = 1000
    grid = N // TN
    return pl.pallas_call(
        _mlp1_body,
        grid=(grid,),
        in_specs=[
            pl.BlockSpec((NC, TN, D), lambda i: (0, i, 0)),
            pl.BlockSpec((TN, D), lambda i: (i, 0)),
            pl.BlockSpec((D, H), lambda i: (0, 0)),
            pl.BlockSpec((1, H), lambda i: (0, 0)),
            pl.BlockSpec((H, D), lambda i: (0, 0)),
            pl.BlockSpec((1, D), lambda i: (0, 0)),
        ],
        out_specs=[
            pl.BlockSpec((TN, D), lambda i: (i, 0)),
            pl.BlockSpec((TN, D), lambda i: (i, 0)),
            pl.BlockSpec((TN, D), lambda i: (i, 0)),
        ],
        out_shape=[
            jax.ShapeDtypeStruct((N, D), F32),
            jax.ShapeDtypeStruct((N, D), F32),
            jax.ShapeDtypeStruct((N, D), F32),
        ],
    )(parts, x, A1, a1.reshape(1, H), A2, a2.reshape(1, D))


def _mlp2_body(ngrid, G, h1_ref, agg1_ref, parts_ref, gid_ref,
               w1_ref, b1_ref, w2_ref, b2_ref,
               p1_ref, pb1_ref, p2_ref, pb2_ref,
               out_ref, pooled_ref):
    i = pl.program_id(0)
    agg2 = agg1_ref[...] + parts_ref[0] + parts_ref[1]
    u = jax.nn.relu(
        jnp.dot(agg2, w1_ref[...], preferred_element_type=F32) + b1_ref[...])
    r2 = jax.nn.relu(
        jnp.dot(u, w2_ref[...], preferred_element_type=F32) + b2_ref[...])
    h2 = h1_ref[...] + r2
    gid = gid_ref[0]
    onehot = (gid == lax.broadcasted_iota(jnp.int32, (G, gid.shape[1]), 0))
    onehot = onehot.astype(F32)

    @pl.when(i == 0)
    def _():
        pooled_ref[...] = jnp.zeros_like(pooled_ref)

    pooled_ref[...] += jnp.dot(onehot, h2, preferred_element_type=F32)

    @pl.when(i == ngrid - 1)
    def _():
        pooled = pooled_ref[...]
        v = jax.nn.relu(
            jnp.dot(pooled, p1_ref[...], preferred_element_type=F32)
            + pb1_ref[...])
        out_ref[...] = (
            jnp.dot(v, p2_ref[...], preferred_element_type=F32) + pb2_ref[...])


def _mlp2_pool(h1, agg1, parts, graph_ids, B1, b1, B2, b2, P1, p1, P2, p2,
               N, D, H, G):
    TN = 1000
    grid = N // TN
    gid3 = graph_ids.reshape(grid, 1, TN)
    return pl.pallas_call(
        functools.partial(_mlp2_body, grid, G),
        grid=(grid,),
        in_specs=[
            pl.BlockSpec((TN, D), lambda i: (i, 0)),
            pl.BlockSpec((TN, D), lambda i: (i, 0)),
            pl.BlockSpec((NC, TN, D), lambda i: (0, i, 0)),
            pl.BlockSpec((1, 1, TN), lambda i: (i, 0, 0)),
            pl.BlockSpec((D, H), lambda i: (0, 0)),
            pl.BlockSpec((1, H), lambda i: (0, 0)),
            pl.BlockSpec((H, D), lambda i: (0, 0)),
            pl.BlockSpec((1, D), lambda i: (0, 0)),
            pl.BlockSpec((D, D), lambda i: (0, 0)),
            pl.BlockSpec((1, D), lambda i: (0, 0)),
            pl.BlockSpec((D, D), lambda i: (0, 0)),
            pl.BlockSpec((1, D), lambda i: (0, 0)),
        ],
        out_specs=pl.BlockSpec((G, D), lambda i: (0, 0)),
        out_shape=jax.ShapeDtypeStruct((G, D), F32),
        scratch_shapes=[pltpu.VMEM((G, D), F32)],
    )(h1, agg1, parts, gid3, B1, b1.reshape(1, H), B2, b2.reshape(1, D),
      P1, p1.reshape(1, D), P2, p2.reshape(1, D))


def kernel(x, edge_attr, We0, be0, A1, a1, A2, a2, B1, b1, B2, b2,
           P1, p1, P2, p2, edge_index, graph_ids):
    N, D = x.shape
    E, DE = edge_attr.shape
    H = A1.shape[1]
    G = 256

    ei_flat = edge_index.reshape(2 * E)
    e = _edge_encoder(edge_attr, We0, be0, E, DE, D)
    parts1 = _make_sc_scatter(N, E, D, with_e=True)(e, x, ei_flat)
    h1, agg1, r1 = _mlp1(parts1, x, A1, a1, A2, a2, N, D, H)
    parts2 = _make_sc_scatter(N, E, D, with_e=False)(r1, ei_flat)
    out = _mlp2_pool(h1, agg1, parts2, graph_ids,
                     B1, b1, B2, b2, P1, p1, P2, p2, N, D, H, G)
    return out
